# baseline (speedup 1.0000x reference)
import functools

import jax
import jax.numpy as jnp
from jax import lax
from jax.experimental import pallas as pl
from jax.experimental.pallas import tpu as pltpu
from jax.experimental.pallas import tpu_sc as plsc

NC = 2
NS = 16
NW = NC * NS
L = 16

B = 64
SEQ = 512
H = 768
HC = H // L
TOK = B * SEQ
TPW = TOK // NW
CH = 16
NCH = TPW // CH
CPR = SEQ // CH
NSET = 4
TG = 8
PAD_ID = 1
MAXPOS = 514
EPS = 1e-5


def _pipeline(ids_ref, word_ref, pos_ref, tt_ref, g_ref, b_ref, out_ref,
              idx_w, idx_p, bufs_a, bufs_b, tt_v, g_v, b_v,
              sems_a, sems_b, sems_o):
  wid = lax.axis_index("s") * NC + lax.axis_index("c")
  pltpu.sync_copy(tt_ref, tt_v)
  pltpu.sync_copy(g_ref, g_v)
  pltpu.sync_copy(b_ref, b_v)
  base = wid * TPW

  def tok0_of(c):
    return base + c * CH

  def prep(c, carry_k, s):
    pltpu.sync_copy(ids_ref.at[pl.ds(tok0_of(c), CH)], idx_w[s])
    carry_k = jnp.where(c % CPR == 0, jnp.int32(0), carry_k)

    def pos_loop(j, k):
      ids16 = idx_w[s][pl.ds(j * L, L)]
      m = jnp.where(ids16 != PAD_ID, jnp.int32(1), jnp.int32(0))
      cs = jnp.cumsum(m) + k
      idx_p[s][pl.ds(j * L, L)] = jnp.minimum(cs * m + 1,
                                              jnp.int32(MAXPOS - 1))
      return jnp.max(cs)

    return lax.fori_loop(0, CH // L, pos_loop, carry_k)

  def gather_a(s):
    return pltpu.make_async_copy(word_ref.at[idx_w[s]], bufs_a[s],
                                 sems_a[s])

  def gather_b(s):
    return pltpu.make_async_copy(pos_ref.at[idx_p[s]], bufs_b[s],
                                 sems_b[s])

  def out_copy(c, s):
    return pltpu.make_async_copy(bufs_a[s],
                                 out_ref.at[pl.ds(tok0_of(c), CH)],
                                 sems_o[s])

  def ln_chunk(s):
    buf_a = bufs_a[s]
    buf_b = bufs_b[s]
    for grp in range(CH // TG):
      t0 = grp * TG

      def p1(j, carry):
        carry = list(carry)
        sl = pl.ds(j * L, L)
        ttj = tt_v[sl]
        for t in range(TG):
          d = buf_a[t0 + t, sl] + buf_b[t0 + t, sl] + ttj
          buf_a[t0 + t, sl] = d
          carry[2 * t] = carry[2 * t] + d
          carry[2 * t + 1] = carry[2 * t + 1] + d * d
        return tuple(carry)

      z = jnp.zeros((L,), jnp.float32)
      carry = lax.fori_loop(0, HC, p1, (z,) * (2 * TG))

      scales = []
      for t in range(TG):
        mean = jnp.sum(carry[2 * t]) * (1.0 / H)
        ex2 = jnp.sum(carry[2 * t + 1]) * (1.0 / H)
        x = (ex2 - mean * mean) + EPS
        i = lax.bitcast_convert_type(x, jnp.int32)
        i = jnp.int32(0x5F3759DF) - lax.shift_right_logical(i, 1)
        y = lax.bitcast_convert_type(i, jnp.float32)
        y = y * (1.5 - 0.5 * x * y * y)
        y = y * (1.5 - 0.5 * x * y * y)
        y = y * (1.5 - 0.5 * x * y * y)
        scales.append((y, mean * y))

      def p2(j, _):
        sl = pl.ds(j * L, L)
        gj = g_v[sl]
        bj = b_v[sl]
        for t in range(TG):
          d = buf_a[t0 + t, sl]
          buf_a[t0 + t, sl] = (d * scales[t][0] - scales[t][1]) * gj + bj
        return 0

      lax.fori_loop(0, HC, p2, 0)

  carry_k = prep(0, jnp.int32(0), 0)
  gather_a(0).start()
  gather_b(0).start()
  carry_k = prep(1, carry_k, 1)
  gather_a(1).start()
  gather_b(1).start()
  out_copy(2, 2).start()
  out_copy(3, 3).start()

  def phase(p, carry_k, s):
    s_next = (s + 2) % NSET
    gather_a(s).wait()
    gather_b(s).wait()
    ln_chunk(s)
    out_copy(p, s).start()
    c_next = jnp.minimum(p + 2, NCH - 1)
    carry_k = prep(c_next, carry_k, s_next)
    pltpu.make_async_copy(
        bufs_a[s_next],
        out_ref.at[pl.ds(tok0_of(jnp.maximum(p - 2, 0)), CH)],
        sems_o[s_next]).wait()
    gather_a(s_next).start()
    gather_b(s_next).start()
    return carry_k

  def body_i(i, carry_k):
    p = NSET * i
    for u in range(NSET):
      carry_k = phase(p + u, carry_k, u)
    return carry_k

  lax.fori_loop(0, NCH // NSET, body_i, carry_k)

  pltpu.make_async_copy(bufs_a[2], out_ref.at[pl.ds(tok0_of(NCH - 2), CH)],
                        sems_o[2]).wait()
  pltpu.make_async_copy(bufs_a[3], out_ref.at[pl.ds(tok0_of(NCH - 1), CH)],
                        sems_o[3]).wait()
  gather_a(0).wait()
  gather_b(0).wait()
  gather_a(1).wait()
  gather_b(1).wait()


@functools.partial(
    pl.kernel,
    out_type=jax.ShapeDtypeStruct((TOK, H), jnp.float32),
    mesh=plsc.VectorSubcoreMesh(
        core_axis_name="c", subcore_axis_name="s",
        num_cores=NC, num_subcores=NS),
    compiler_params=pltpu.CompilerParams(needs_layout_passes=False),
    scratch_types=(
        [pltpu.VMEM((CH,), jnp.int32) for _ in range(NSET)]
        + [pltpu.VMEM((CH,), jnp.int32) for _ in range(NSET)]
        + [pltpu.VMEM((CH, H), jnp.float32) for _ in range(NSET)]
        + [pltpu.VMEM((CH, H), jnp.float32) for _ in range(NSET)]
        + [pltpu.VMEM((H,), jnp.float32) for _ in range(3)]
        + [pltpu.SemaphoreType.DMA for _ in range(3 * NSET)]
    ),
)
def _sc_embed_ln(ids_ref, word_ref, pos_ref, tt_ref, g_ref, b_ref, out_ref,
                 *scratch):
  idx_w = list(scratch[0:NSET])
  idx_p = list(scratch[NSET:2 * NSET])
  bufs_a = list(scratch[2 * NSET:3 * NSET])
  bufs_b = list(scratch[3 * NSET:4 * NSET])
  tt_v, g_v, b_v = scratch[4 * NSET:4 * NSET + 3]
  sems = scratch[4 * NSET + 3:]
  sems_a = list(sems[0:NSET])
  sems_b = list(sems[NSET:2 * NSET])
  sems_o = list(sems[2 * NSET:3 * NSET])
  _pipeline(ids_ref, word_ref, pos_ref, tt_ref, g_ref, b_ref, out_ref,
            idx_w, idx_p, bufs_a, bufs_b, tt_v, g_v, b_v,
            sems_a, sems_b, sems_o)


@jax.jit
def kernel(input_ids, word_embeddings, token_type_embeddings,
           position_embeddings, ln_gamma, ln_beta):
  ids = input_ids.reshape(TOK).astype(jnp.int32)
  tt_row = token_type_embeddings.reshape(H)
  out = _sc_embed_ln(ids, word_embeddings, position_embeddings,
                     tt_row, ln_gamma, ln_beta)
  return out.reshape(B, SEQ, H)
